# baseline (speedup 1.0000x reference)
import functools

import jax
import jax.numpy as jnp
from jax import lax
from jax.experimental import pallas as pl
from jax.experimental.pallas import tpu as pltpu
from jax.experimental.pallas import tpu_sc as plsc

N = 10000
D = 128
DH = D // 2
W1 = 80
W2 = 16
HID = 200
E = 320000
F32 = jnp.float32

NP = 10240
C = 128
NC, NS = 2, 16
NW = NC * NS
EP = 327680
NCHUNK = EP // C
CH1 = NCHUNK // NS
CH2 = NCHUNK // NW
RPT = NP // NS
ZB = RPT // C
NSLOT = 5

_mesh = plsc.VectorSubcoreMesh(core_axis_name="c", subcore_axis_name="s")


def _ring(nchunks, ec, chunk0, table, acc_s, epair, rows, isem, gsem, ssem):
    def idx(ci, b):
        pltpu.async_copy(ec.at[pl.ds(2 * (chunk0 + ci), 2), :], epair[b], isem[b])

    def gather(ci, b):
        pltpu.make_async_copy(ec.at[pl.ds(2 * (chunk0 + ci), 2), :], epair[b],
                              isem[b]).wait()
        pltpu.async_copy(table.at[epair[b].at[0]], rows[b], gsem[b])

    def drain_scatter(b):
        pltpu.make_async_copy(rows[b], acc_s.at[epair[b].at[1]], ssem[b]).wait()

    def visit(k, j, drain):
        b = j % NSLOT
        pltpu.make_async_copy(table.at[epair[b].at[0]], rows[b], gsem[b]).wait()
        pltpu.async_copy(rows[b], acc_s.at[epair[b].at[1]], ssem[b], add=True)
        if not isinstance(k, int) or k + 3 < nchunks:
            if drain:
                drain_scatter((j + 3) % NSLOT)
            idx(k + 3, (j + 3) % NSLOT)
        if not isinstance(k, int) or k + 2 < nchunks:
            gather(k + 2, (j + 2) % NSLOT)

    for k in range(NSLOT):
        visit(k, k, k >= 2)

    @pl.loop(NSLOT, nchunks - NSLOT, step=NSLOT)
    def _(k0):
        for j in range(NSLOT):
            visit(k0 + j, j, True)

    for k in range(nchunks - NSLOT, nchunks):
        visit(k, k % NSLOT, k + 3 < nchunks)
    for b in range(NSLOT):
        drain_scatter(b)


def _prologue(ec, chunk0, table, zrows, acc_s, r0, epair, rows, isem, gsem):
    for j in range(3):
        pltpu.async_copy(ec.at[pl.ds(2 * (chunk0 + j), 2), :], epair[j], isem[j])
    pltpu.sync_copy(zrows, rows[0])

    @pl.loop(0, ZB)
    def _(j):
        pltpu.sync_copy(rows[0], acc_s.at[pl.ds(r0 + j * C, C), :])

    for j in range(2):
        pltpu.make_async_copy(ec.at[pl.ds(2 * (chunk0 + j), 2), :], epair[j],
                              isem[j]).wait()
        pltpu.async_copy(table.at[epair[j].at[0]], rows[j], gsem[j])


@functools.partial(
    pl.kernel,
    out_type=[jax.ShapeDtypeStruct((NP, W1), F32),
              jax.ShapeDtypeStruct((NP, W1), F32)],
    mesh=_mesh,
    compiler_params=pltpu.CompilerParams(use_tc_tiling_on_sc=False),
    scratch_types=[[pltpu.VMEM((2, C), jnp.int32)] * NSLOT,
                   [pltpu.VMEM((C, W1), F32)] * NSLOT,
                   pltpu.VMEM_SHARED((NP, W1), F32),
                   [pltpu.SemaphoreType.DMA] * NSLOT,
                   [pltpu.SemaphoreType.DMA] * NSLOT,
                   [pltpu.SemaphoreType.DMA] * NSLOT])
def _sc_l1(xcat, epairs, zrows,
           aggL, aggR,
           epair, rows, acc_s, isem, gsem, ssem):
    c = lax.axis_index("c")
    s = lax.axis_index("s")
    r0 = s * RPT
    ec = epairs.at[c]
    chunk0 = s * CH1

    _prologue(ec, chunk0, xcat, zrows, acc_s, r0, epair, rows, isem, gsem)
    plsc.subcore_barrier()
    _ring(CH1, ec, chunk0, xcat, acc_s, epair, rows, isem, gsem, ssem)
    plsc.subcore_barrier()

    @pl.when(c == 0)
    def _():
        pltpu.sync_copy(acc_s.at[pl.ds(r0, RPT), :], aggL.at[pl.ds(r0, RPT), :])

    @pl.when(c == 1)
    def _():
        pltpu.sync_copy(acc_s.at[pl.ds(r0, RPT), :], aggR.at[pl.ds(r0, RPT), :])


@functools.partial(
    pl.kernel,
    out_type=[jax.ShapeDtypeStruct((NP, W2), F32),
              jax.ShapeDtypeStruct((NP, W2), F32)],
    mesh=_mesh,
    compiler_params=pltpu.CompilerParams(use_tc_tiling_on_sc=False),
    scratch_types=[[pltpu.VMEM((2, C), jnp.int32)] * NSLOT,
                   [pltpu.VMEM((C, W2), F32)] * NSLOT,
                   pltpu.VMEM_SHARED((NP, W2), F32),
                   [pltpu.SemaphoreType.DMA] * NSLOT,
                   [pltpu.SemaphoreType.DMA] * NSLOT,
                   [pltpu.SemaphoreType.DMA] * NSLOT])
def _sc_l2(z16, epairs, zrows,
           sumA, sumB,
           epair, rows, acc_s, isem, gsem, ssem):
    c = lax.axis_index("c")
    s = lax.axis_index("s")
    r0 = s * RPT
    ec = epairs.at[0]
    chunk0 = (c * NS + s) * CH2

    _prologue(ec, chunk0, z16, zrows, acc_s, r0, epair, rows, isem, gsem)
    plsc.subcore_barrier()
    _ring(CH2, ec, chunk0, z16, acc_s, epair, rows, isem, gsem, ssem)
    plsc.subcore_barrier()

    @pl.when(c == 0)
    def _():
        pltpu.sync_copy(acc_s.at[pl.ds(r0, RPT), :], sumA.at[pl.ds(r0, RPT), :])

    @pl.when(c == 1)
    def _():
        pltpu.sync_copy(acc_s.at[pl.ds(r0, RPT), :], sumB.at[pl.ds(r0, RPT), :])


BLK = 2000
_G = N // BLK


def _tc1_body(aggL, aggR, xr, wl1tL, wl1tR, wr1t, bl1, wl2t, wr2t, bl2,
              z_out, r2b_out, inv_out):
    inv = 1.0 / jnp.maximum(aggL[..., DH:DH + 1], 1.0)
    h = (jnp.dot(aggL[..., :DH] * inv, wl1tL[...], preferred_element_type=F32)
         + jnp.dot(aggR[..., :DH] * inv, wl1tR[...], preferred_element_type=F32)
         + bl1[...]
         + jnp.dot(xr[...], wr1t[...], preferred_element_type=F32))
    h = jnp.maximum(h, 0.0)
    z_out[...] = jnp.dot(h, wl2t[...], preferred_element_type=F32)
    r2b_out[...] = jnp.dot(h, wr2t[...], preferred_element_type=F32) + bl2[...]
    inv_out[...] = inv


def _tc2_body(a2A, a2B, inv, r2b, out):
    m2 = (a2A[..., :2] + a2B[..., :2]) * inv[...]
    out[...] = jnp.maximum(m2 + r2b[...], 0.0)


def kernel(x, edge_index, Wl1, bl1, Wr1, Wl2, bl2, Wr2):
    xcat = (jnp.zeros((2 * NP, W1), F32)
            .at[:N, :DH].set(x[:, :DH])
            .at[NP:NP + N, :DH].set(x[:, DH:])
            .at[:, DH].set(1.0))
    src = jnp.concatenate([edge_index[0], jnp.zeros((EP - E,), jnp.int32)])
    dst = jnp.concatenate([edge_index[1], jnp.full((EP - E,), N, jnp.int32)])
    src3 = src.reshape(NCHUNK, C)
    dst3 = dst.reshape(NCHUNK, C)
    epairs = jnp.stack([jnp.stack([src3, dst3], axis=1),
                        jnp.stack([src3 + NP, dst3], axis=1)]
                       ).reshape(2, 2 * NCHUNK, C)
    zrows1 = jnp.zeros((C, W1), F32)
    zrows2 = jnp.zeros((C, W2), F32)

    aggL, aggR = _sc_l1(xcat, epairs, zrows1)

    wl1t = Wl1.T
    wl2t = jnp.zeros((HID, W2), F32).at[:, :2].set(Wl2.T)
    row_spec = lambda w: pl.BlockSpec((BLK, w), lambda i: (i, 0))
    full_spec = lambda a, b: pl.BlockSpec((a, b), lambda i: (0, 0))

    z16, r2b, inv = pl.pallas_call(
        _tc1_body,
        grid=(_G,),
        in_specs=[row_spec(W1), row_spec(W1), row_spec(D),
                  full_spec(DH, HID), full_spec(DH, HID), full_spec(D, HID),
                  full_spec(1, HID), full_spec(HID, W2), full_spec(HID, 2),
                  full_spec(1, 2)],
        out_specs=[row_spec(W2), row_spec(2), row_spec(1)],
        out_shape=[jax.ShapeDtypeStruct((N, W2), F32),
                   jax.ShapeDtypeStruct((N, 2), F32),
                   jax.ShapeDtypeStruct((N, 1), F32)],
    )(aggL, aggR, x, wl1t[:DH], wl1t[DH:], Wr1.T,
      bl1.reshape(1, HID), wl2t, Wr2.T, bl2.reshape(1, 2))

    a2A, a2B = _sc_l2(z16, epairs, zrows2)

    out = pl.pallas_call(
        _tc2_body,
        grid=(_G,),
        in_specs=[row_spec(W2), row_spec(W2), row_spec(1), row_spec(2)],
        out_specs=row_spec(2),
        out_shape=jax.ShapeDtypeStruct((N, 2), F32),
    )(a2A, a2B, inv, r2b)

    return out


# speedup vs baseline: 1.3322x; 1.3322x over previous
import functools

import jax
import jax.numpy as jnp
from jax import lax
from jax.experimental import pallas as pl
from jax.experimental.pallas import tpu as pltpu
from jax.experimental.pallas import tpu_sc as plsc

N = 10000
D = 128
DH = D // 2
W1 = 96
W2 = 16
HID = 200
E = 320000
F32 = jnp.float32
BF16 = jnp.bfloat16

NP = 10240
C = 128
NC, NS = 2, 16
NW = NC * NS
EP = 327680
NCHUNK = EP // C
CH1 = NCHUNK // NS
CH2 = NCHUNK // NW
RPT = NP // NS
ZB = RPT // C
NSLOT = 5

_mesh = plsc.VectorSubcoreMesh(core_axis_name="c", subcore_axis_name="s")


def _ring(nchunks, ec, chunk0, table, acc_s, epair, rows, isem, gsem, ssem):
    def idx(ci, b):
        pltpu.async_copy(ec.at[chunk0 + ci], epair[b], isem[b])

    def gather(ci, b):
        pltpu.make_async_copy(ec.at[chunk0 + ci], epair[b], isem[b]).wait()
        pltpu.async_copy(table.at[epair[b].at[0]], rows[b], gsem[b])

    def drain_scatter(b):
        pltpu.make_async_copy(rows[b], acc_s.at[epair[b].at[1]], ssem[b]).wait()

    def visit(k, j, drain):
        b = j % NSLOT
        pltpu.make_async_copy(table.at[epair[b].at[0]], rows[b], gsem[b]).wait()
        pltpu.async_copy(rows[b], acc_s.at[epair[b].at[1]], ssem[b], add=True)
        if not isinstance(k, int) or k + 3 < nchunks:
            if drain:
                drain_scatter((j + 3) % NSLOT)
            idx(k + 3, (j + 3) % NSLOT)
        if not isinstance(k, int) or k + 2 < nchunks:
            gather(k + 2, (j + 2) % NSLOT)

    for k in range(NSLOT):
        visit(k, k, k >= 2)

    @pl.loop(NSLOT, nchunks - NSLOT, step=NSLOT)
    def _(k0):
        for j in range(NSLOT):
            visit(k0 + j, j, True)

    for k in range(nchunks - NSLOT, nchunks):
        visit(k, k % NSLOT, k + 3 < nchunks)
    for b in range(NSLOT):
        drain_scatter(b)


def _prologue(ec, chunk0, table, zrows, acc_s, r0, epair, rows, isem, gsem):
    for j in range(3):
        pltpu.async_copy(ec.at[chunk0 + j], epair[j], isem[j])
    pltpu.sync_copy(zrows, rows[0])

    @pl.loop(0, ZB)
    def _(j):
        pltpu.sync_copy(rows[0], acc_s.at[pl.ds(r0 + j * C, C), :])

    for j in range(2):
        pltpu.make_async_copy(ec.at[chunk0 + j], epair[j], isem[j]).wait()
        pltpu.async_copy(table.at[epair[j].at[0]], rows[j], gsem[j])


@functools.partial(
    pl.kernel,
    out_type=[jax.ShapeDtypeStruct((NP, W1), BF16),
              jax.ShapeDtypeStruct((NP, W1), BF16)],
    mesh=_mesh,
    compiler_params=pltpu.CompilerParams(use_tc_tiling_on_sc=False),
    scratch_types=[[pltpu.VMEM((2, C), jnp.int32)] * NSLOT,
                   [pltpu.VMEM((C, W1), BF16)] * NSLOT,
                   pltpu.VMEM_SHARED((NP, W1), BF16),
                   [pltpu.SemaphoreType.DMA] * NSLOT,
                   [pltpu.SemaphoreType.DMA] * NSLOT,
                   [pltpu.SemaphoreType.DMA] * NSLOT])
def _sc_l1(xcat, epairs, zrows,
           aggL, aggR,
           epair, rows, acc_s, isem, gsem, ssem):
    c = lax.axis_index("c")
    s = lax.axis_index("s")
    r0 = s * RPT
    ec = epairs.at[c]
    chunk0 = s * CH1

    _prologue(ec, chunk0, xcat, zrows, acc_s, r0, epair, rows, isem, gsem)
    plsc.subcore_barrier()
    _ring(CH1, ec, chunk0, xcat, acc_s, epair, rows, isem, gsem, ssem)
    plsc.subcore_barrier()

    @pl.when(c == 0)
    def _():
        pltpu.sync_copy(acc_s.at[pl.ds(r0, RPT), :], aggL.at[pl.ds(r0, RPT), :])

    @pl.when(c == 1)
    def _():
        pltpu.sync_copy(acc_s.at[pl.ds(r0, RPT), :], aggR.at[pl.ds(r0, RPT), :])


@functools.partial(
    pl.kernel,
    out_type=[jax.ShapeDtypeStruct((NP, W2), F32),
              jax.ShapeDtypeStruct((NP, W2), F32)],
    mesh=_mesh,
    compiler_params=pltpu.CompilerParams(use_tc_tiling_on_sc=False),
    scratch_types=[[pltpu.VMEM((2, C), jnp.int32)] * NSLOT,
                   [pltpu.VMEM((C, W2), F32)] * NSLOT,
                   pltpu.VMEM_SHARED((NP, W2), F32),
                   [pltpu.SemaphoreType.DMA] * NSLOT,
                   [pltpu.SemaphoreType.DMA] * NSLOT,
                   [pltpu.SemaphoreType.DMA] * NSLOT])
def _sc_l2(z16, epairs, zrows,
           sumA, sumB,
           epair, rows, acc_s, isem, gsem, ssem):
    c = lax.axis_index("c")
    s = lax.axis_index("s")
    r0 = s * RPT
    ec = epairs.at[0]
    chunk0 = (c * NS + s) * CH2

    _prologue(ec, chunk0, z16, zrows, acc_s, r0, epair, rows, isem, gsem)
    plsc.subcore_barrier()
    _ring(CH2, ec, chunk0, z16, acc_s, epair, rows, isem, gsem, ssem)
    plsc.subcore_barrier()

    @pl.when(c == 0)
    def _():
        pltpu.sync_copy(acc_s.at[pl.ds(r0, RPT), :], sumA.at[pl.ds(r0, RPT), :])

    @pl.when(c == 1)
    def _():
        pltpu.sync_copy(acc_s.at[pl.ds(r0, RPT), :], sumB.at[pl.ds(r0, RPT), :])


BLK = 2048
_G = NP // BLK


def _tc1_body(aggL, aggR, xr, wl1tL, wl1tR, wr1t, bl1, wl2t, wr2t, bl2,
              z_out, r2b_out, inv_out):
    inv = 1.0 / jnp.maximum(aggL[..., DH:DH + 1].astype(F32), 1.0)
    aL = aggL[..., :DH].astype(F32) * inv
    aR = aggR[..., :DH].astype(F32) * inv
    h = (jnp.dot(aL, wl1tL[...], preferred_element_type=F32)
         + jnp.dot(aR, wl1tR[...], preferred_element_type=F32)
         + bl1[...]
         + jnp.dot(xr[...], wr1t[...], preferred_element_type=F32))
    h = jnp.maximum(h, 0.0)
    i = pl.program_id(0)
    rows = lax.broadcasted_iota(jnp.int32, (BLK, 1), 0) + i * BLK
    z = jnp.dot(h, wl2t[...], preferred_element_type=F32)
    z_out[...] = jnp.where(rows < N, z, 0.0)
    r2b_out[...] = jnp.dot(h, wr2t[...], preferred_element_type=F32) + bl2[...]
    inv_out[...] = inv


def _tc2_body(a2A, a2B, inv, r2b, out):
    m2 = (a2A[..., :2] + a2B[..., :2]) * inv[...]
    out[...] = jnp.maximum(m2 + r2b[...], 0.0)


def kernel(x, edge_index, Wl1, bl1, Wr1, Wl2, bl2, Wr2):
    xp = jnp.zeros((NP, D), F32).at[:N].set(x)
    xcat = (jnp.zeros((2 * NP, W1), BF16)
            .at[:NP, :DH].set(xp[:, :DH].astype(BF16))
            .at[NP:, :DH].set(xp[:, DH:].astype(BF16))
            .at[:, DH].set(1.0))
    src = jnp.concatenate([edge_index[0], jnp.zeros((EP - E,), jnp.int32)])
    dst = jnp.concatenate([edge_index[1], jnp.full((EP - E,), N, jnp.int32)])
    src3 = src.reshape(NCHUNK, C)
    dst3 = dst.reshape(NCHUNK, C)
    epairs = jnp.stack([jnp.stack([src3, dst3], axis=1),
                        jnp.stack([src3 + NP, dst3], axis=1)])
    zrows1 = jnp.zeros((C, W1), BF16)
    zrows2 = jnp.zeros((C, W2), F32)

    aggL, aggR = _sc_l1(xcat, epairs, zrows1)

    wl1t = Wl1.T
    wl2t = jnp.zeros((HID, W2), F32).at[:, :2].set(Wl2.T)
    row_spec = lambda w: pl.BlockSpec((BLK, w), lambda i: (i, 0))
    full_spec = lambda a, b: pl.BlockSpec((a, b), lambda i: (0, 0))

    z16, r2b, inv = pl.pallas_call(
        _tc1_body,
        grid=(_G,),
        in_specs=[row_spec(W1), row_spec(W1), row_spec(D),
                  full_spec(DH, HID), full_spec(DH, HID), full_spec(D, HID),
                  full_spec(1, HID), full_spec(HID, W2), full_spec(HID, 2),
                  full_spec(1, 2)],
        out_specs=[row_spec(W2), row_spec(2), row_spec(1)],
        out_shape=[jax.ShapeDtypeStruct((NP, W2), F32),
                   jax.ShapeDtypeStruct((NP, 2), F32),
                   jax.ShapeDtypeStruct((NP, 1), F32)],
    )(aggL, aggR, xp, wl1t[:DH], wl1t[DH:], Wr1.T,
      bl1.reshape(1, HID), wl2t, Wr2.T, bl2.reshape(1, 2))

    a2A, a2B = _sc_l2(z16, epairs, zrows2)

    out = pl.pallas_call(
        _tc2_body,
        grid=(_G,),
        in_specs=[row_spec(W2), row_spec(W2), row_spec(1), row_spec(2)],
        out_specs=row_spec(2),
        out_shape=jax.ShapeDtypeStruct((NP, 2), F32),
    )(a2A, a2B, inv, r2b)

    return out[:N]
